# baseline (speedup 1.0000x reference)
import functools

import jax
import jax.numpy as jnp
from jax.experimental import pallas as pl

_EPS = 1e-05
_TOP_BITS = 0x40000000
_BIG_I32 = 2**31 - 1
_RB = 16


def _rows_kernel(lref, gref, tref, kref, pref, mpref, oref):
    x = lref[...]
    R = x.shape[1]
    pos = (jax.lax.broadcasted_iota(jnp.int32, (1, R, 128), 1) * 128
           + jax.lax.broadcasted_iota(jnp.int32, (1, R, 128), 2))

    m0 = jnp.max(x, axis=(1, 2), keepdims=True)
    gidx = jnp.min(jnp.where(x == m0, pos, _BIG_I32), axis=(1, 2))

    t = tref[...][:, :1][:, :, None]
    kk = kref[...][:, :1][:, :, None]
    pp = pref[...][:, :1][:, :, None]
    mp = mpref[...][:, :1][:, :, None]

    tp = jnp.where(t < _EPS, 1.0, t)
    sl = x / tp
    m1 = m0 / tp
    e = jnp.exp(sl - m1)
    e2 = jnp.where(e >= mp, e, 0.0)
    z2 = jnp.sum(e2, axis=(1, 2), keepdims=True)
    target = (1.0 - pp) * z2

    def body(_, carry):
        lok, hik, lop, hip = carry
        midk = (lok + hik + 1) // 2
        midp = (lop + hip + 1) // 2
        xk = jax.lax.bitcast_convert_type(midk, jnp.float32)
        xp = jax.lax.bitcast_convert_type(midp, jnp.float32)
        cnt = jnp.sum(jnp.where(e2 >= xk, 1.0, 0.0), axis=(1, 2), keepdims=True)
        mass = jnp.sum(jnp.where(e2 < xp, e2, 0.0), axis=(1, 2), keepdims=True)
        okk = cnt >= kk
        okp = mass <= target
        lok = jnp.where(okk, midk, lok)
        hik = jnp.where(okk, hik, midk - 1)
        lop = jnp.where(okp, midp, lop)
        hip = jnp.where(okp, hip, midp - 1)
        return lok, hik, lop, hip

    zero = jnp.zeros((_RB, 1, 1), jnp.int32)
    top = jnp.full((_RB, 1, 1), _TOP_BITS, jnp.int32)
    lok, _, lop, _ = jax.lax.fori_loop(0, 30, body, (zero, top, zero, top))

    tk = jax.lax.bitcast_convert_type(lok, jnp.float32)
    bp = jax.lax.bitcast_convert_type(lop, jnp.float32)
    vstar = jnp.max(jnp.where(e2 <= bp, e2, 0.0), axis=(1, 2), keepdims=True)

    kept = jnp.logical_and(e2 >= tk, e2 >= vstar)
    score = jnp.where(kept, sl + gref[...], -jnp.inf)
    ms = jnp.max(score, axis=(1, 2), keepdims=True)
    ridx = jnp.min(jnp.where(score == ms, pos, _BIG_I32), axis=(1, 2))

    samp = jnp.where(t[:, 0, 0] < _EPS, gidx, ridx)
    oref[...] = jnp.broadcast_to(samp[:, None, None], (_RB, 1, 128))


@jax.jit
def _run(lp, gp, temperature, top_k, top_p, min_p):
    B, R, _ = lp.shape
    sc = pl.BlockSpec((_RB, 128), lambda i: (i, 0))
    out = pl.pallas_call(
        _rows_kernel,
        grid=(B // _RB,),
        in_specs=[
            pl.BlockSpec((_RB, R, 128), lambda i: (i, 0, 0)),
            pl.BlockSpec((_RB, R, 128), lambda i: (i, 0, 0)),
            sc, sc, sc, sc,
        ],
        out_specs=pl.BlockSpec((_RB, 1, 128), lambda i: (i, 0, 0)),
        out_shape=jax.ShapeDtypeStruct((B, 1, 128), jnp.int32),
    )(lp, gp, temperature, top_k, top_p, min_p)
    return out[:, 0, :1]


@functools.cache
def _gumbel_padded(B, V, Vp):
    g = jax.random.gumbel(jax.random.key(1234), (B, V), dtype=jnp.float32)
    gp = jnp.pad(g, ((0, 0), (0, Vp - V))).reshape(B, Vp // 128, 128)
    return jax.block_until_ready(gp)


def kernel(logits, temperature, top_k, top_p, min_p):
    logits = logits.astype(jnp.float32)
    B, V = logits.shape
    Vp = ((V + 1023) // 1024) * 1024
    R = Vp // 128
    lp = jnp.pad(logits, ((0, 0), (0, Vp - V)), constant_values=-jnp.inf)
    gp = _gumbel_padded(B, V, Vp)
    lp = lp.reshape(B, R, 128)
    tb = jnp.broadcast_to(temperature[:, None], (B, 128))
    kb = jnp.broadcast_to(top_k.astype(jnp.float32)[:, None], (B, 128))
    pb = jnp.broadcast_to(top_p[:, None], (B, 128))
    mb = jnp.broadcast_to(min_p[:, None], (B, 128))
    return _run(lp, gp, tb, kb, pb, mb)
